# Initial kernel scaffold; baseline (speedup 1.0000x reference)
import functools

import jax
import jax.numpy as jnp
from jax import lax
from jax.experimental import pallas as pl
from jax.experimental.pallas import tpu as pltpu
from jax.experimental.pallas import tpu_sc as plsc

N_NODES = 10000
F = 128
NC = 2
NS = 16
LANES = 16
NW = NC * NS
K = 128
ROWS_PER_TILE = N_NODES // NS


def _spmm_partials(x2d, rows3, cols3, lv3, zeros2d):
  nch = rows3.shape[1]
  mesh = plsc.VectorSubcoreMesh(core_axis_name="c", subcore_axis_name="s")

  @functools.partial(
      pl.kernel,
      out_type=jax.ShapeDtypeStruct((NC, N_NODES, F), jnp.float32),
      mesh=mesh,
      scratch_types=[
          pltpu.VMEM_SHARED((N_NODES, F), jnp.float32),
          pltpu.VMEM((nch, K), jnp.int32),
          pltpu.VMEM((nch, K), jnp.int32),
          pltpu.VMEM((nch, K), jnp.float32),
          pltpu.VMEM((K, F), jnp.float32),
          pltpu.SemaphoreType.DMA,
      ],
  )
  def spmm(x_hbm, rows_hbm, cols_hbm, lv_hbm, z_hbm, out_hbm,
           acc_sh, rows_v, cols_v, lv_v, xbuf, gsem):
    cid = lax.axis_index("c")
    sid = lax.axis_index("s")
    wid = sid * NC + cid

    base = sid * ROWS_PER_TILE
    pltpu.sync_copy(z_hbm.at[pl.ds(base, ROWS_PER_TILE)],
                    acc_sh.at[pl.ds(base, ROWS_PER_TILE)])
    pltpu.sync_copy(rows_hbm.at[wid], rows_v)
    pltpu.sync_copy(cols_hbm.at[wid], cols_v)
    pltpu.sync_copy(lv_hbm.at[wid], lv_v)
    plsc.subcore_barrier()

    def chunk(j, carry):
      pltpu.async_copy(x_hbm.at[cols_v.at[j]], xbuf, gsem).wait()

      def edge(i, c2):
        lv = plsc.load_gather(
            lv_v, [lax.broadcast(j, (LANES,)), lax.broadcast(i, (LANES,))])
        for f in range(F // LANES):
          sl = pl.ds(f * LANES, LANES)
          xbuf[i, sl] = xbuf[i, sl] * lv
        return c2

      lax.fori_loop(0, K, edge, 0)

      pltpu.sync_copy(xbuf, acc_sh.at[rows_v.at[j]], add=True)
      return carry

    lax.fori_loop(0, nch, chunk, 0)
    plsc.subcore_barrier()
    pltpu.sync_copy(acc_sh.at[pl.ds(base, ROWS_PER_TILE)],
                    out_hbm.at[cid, pl.ds(base, ROWS_PER_TILE)])

  return spmm(x2d, rows3, cols3, lv3, zeros2d)


def _merge(parts):
  rb = 2000

  def body(p_ref, o_ref):
    o_ref[...] = p_ref[0] + p_ref[1]

  return pl.pallas_call(
      body,
      out_shape=jax.ShapeDtypeStruct((N_NODES, F), jnp.float32),
      grid=(N_NODES // rb,),
      in_specs=[pl.BlockSpec((NC, rb, F), lambda i: (0, i, 0))],
      out_specs=pl.BlockSpec((rb, F), lambda i: (i, 0)),
  )(parts)


def _finish(x2d, t1, qparts, wa, wb, wc, bias):
  rb = 2000

  def body(x_ref, t1_ref, q_ref, wa_ref, wb_ref, wc_ref, b_ref, o_ref):
    xb = x_ref[...]
    t2 = 2.0 * (q_ref[0] + q_ref[1]) - xb
    y = (jnp.dot(xb, wa_ref[...], preferred_element_type=jnp.float32)
         + jnp.dot(t1_ref[...], wb_ref[...], preferred_element_type=jnp.float32)
         + jnp.dot(t2, wc_ref[...], preferred_element_type=jnp.float32)
         + b_ref[...])
    o_ref[...] = jnp.where(y > 0.0, y, jnp.exp(jnp.minimum(y, 0.0)) - 1.0)

  wspec = pl.BlockSpec((F, F), lambda i: (0, 0))
  return pl.pallas_call(
      body,
      out_shape=jax.ShapeDtypeStruct((N_NODES, F), jnp.float32),
      grid=(N_NODES // rb,),
      in_specs=[
          pl.BlockSpec((rb, F), lambda i: (i, 0)),
          pl.BlockSpec((rb, F), lambda i: (i, 0)),
          pl.BlockSpec((NC, rb, F), lambda i: (0, i, 0)),
          wspec, wspec, wspec,
          pl.BlockSpec((1, F), lambda i: (0, 0)),
      ],
      out_specs=pl.BlockSpec((rb, F), lambda i: (i, 0)),
  )(x2d, t1, qparts, wa, wb, wc, bias)


def kernel(x, edge_index, L_vals, W, b):
  x2d = x.reshape(N_NODES, F)
  e = edge_index.shape[1]
  per_tile = -(-e // NW)
  nch = -(-per_tile // K)
  epad = NW * nch * K
  pad = epad - e
  rows = jnp.concatenate([edge_index[0], jnp.zeros((pad,), jnp.int32)])
  cols = jnp.concatenate([edge_index[1], jnp.zeros((pad,), jnp.int32)])
  lv = jnp.concatenate([L_vals, jnp.zeros((pad,), jnp.float32)])
  rows3 = rows.reshape(NW, nch, K)
  cols3 = cols.reshape(NW, nch, K)
  lv3 = lv.reshape(NW, nch, K)
  zeros2d = jnp.zeros((N_NODES, F), jnp.float32)

  p1 = _spmm_partials(x2d, rows3, cols3, lv3, zeros2d)
  t1 = _merge(p1)
  q = _spmm_partials(t1, rows3, cols3, lv3, zeros2d)

  wp = W.reshape(F, 3, F)
  out = _finish(x2d, t1, q, wp[:, 0, :], wp[:, 1, :], wp[:, 2, :],
                b.reshape(1, F))
  return out.reshape(1, N_NODES, F)


# speedup vs baseline: 3.2351x; 3.2351x over previous
import functools

import jax
import jax.numpy as jnp
from jax import lax
from jax.experimental import pallas as pl
from jax.experimental.pallas import tpu as pltpu
from jax.experimental.pallas import tpu_sc as plsc

N_NODES = 10000
F = 128
NC = 2
NS = 16
LANES = 16
NW = NC * NS
K = 128
NPAD = 10240
ROWS_PER_TILE = NPAD // NS


def _spmm_partials(x2d, rows3, cols3, lv3, zeros2d):
  nch = rows3.shape[1]
  mesh = plsc.VectorSubcoreMesh(core_axis_name="c", subcore_axis_name="s")

  @functools.partial(
      pl.kernel,
      out_type=jax.ShapeDtypeStruct((NC, NPAD, F), jnp.float32),
      mesh=mesh,
      scratch_types=[
          pltpu.VMEM_SHARED((NPAD, F), jnp.float32),
          pltpu.VMEM((nch, K), jnp.int32),
          pltpu.VMEM((nch, K), jnp.int32),
          pltpu.VMEM((K * LANES,), jnp.float32),
          pltpu.VMEM((K, F), jnp.float32),
          pltpu.SemaphoreType.DMA,
          pltpu.SemaphoreType.DMA,
      ],
  )
  def spmm(x_hbm, rows_hbm, cols_hbm, lv_hbm, z_hbm, out_hbm,
           acc_sh, rows_v, cols_v, lv_v, xbuf, gsem, lsem):
    cid = lax.axis_index("c")
    sid = lax.axis_index("s")
    wid = sid * NC + cid

    base = sid * ROWS_PER_TILE
    pltpu.sync_copy(z_hbm.at[pl.ds(base, ROWS_PER_TILE)],
                    acc_sh.at[pl.ds(base, ROWS_PER_TILE)])
    pltpu.sync_copy(rows_hbm.at[wid], rows_v)
    pltpu.sync_copy(cols_hbm.at[wid], cols_v)
    plsc.subcore_barrier()

    def chunk(j, carry):
      gcp = pltpu.async_copy(x_hbm.at[cols_v.at[j]], xbuf, gsem)
      pltpu.async_copy(lv_hbm.at[wid, j], lv_v, lsem).wait()
      gcp.wait()

      def edge(i, c2):
        lv = lv_v[pl.ds(i * LANES, LANES)]
        for f in range(F // LANES):
          sl = pl.ds(f * LANES, LANES)
          xbuf[i, sl] = xbuf[i, sl] * lv
        return c2

      lax.fori_loop(0, K, edge, 0)

      pltpu.sync_copy(xbuf, acc_sh.at[rows_v.at[j]], add=True)
      return carry

    lax.fori_loop(0, nch, chunk, 0)
    plsc.subcore_barrier()
    pltpu.sync_copy(acc_sh.at[pl.ds(base, ROWS_PER_TILE)],
                    out_hbm.at[cid, pl.ds(base, ROWS_PER_TILE)])

  return spmm(x2d, rows3, cols3, lv3, zeros2d)


def _merge(parts):
  rb = 2048

  def body(p_ref, o_ref):
    o_ref[...] = p_ref[0] + p_ref[1]

  return pl.pallas_call(
      body,
      out_shape=jax.ShapeDtypeStruct((NPAD, F), jnp.float32),
      grid=(NPAD // rb,),
      in_specs=[pl.BlockSpec((NC, rb, F), lambda i: (0, i, 0))],
      out_specs=pl.BlockSpec((rb, F), lambda i: (i, 0)),
  )(parts)


def _finish(x2d, t1, qparts, wa, wb, wc, bias):
  rb = 2000

  def body(x_ref, t1_ref, q_ref, wa_ref, wb_ref, wc_ref, b_ref, o_ref):
    xb = x_ref[...]
    t2 = 2.0 * (q_ref[0] + q_ref[1]) - xb
    y = (jnp.dot(xb, wa_ref[...], preferred_element_type=jnp.float32)
         + jnp.dot(t1_ref[...], wb_ref[...], preferred_element_type=jnp.float32)
         + jnp.dot(t2, wc_ref[...], preferred_element_type=jnp.float32)
         + b_ref[...])
    o_ref[...] = jnp.where(y > 0.0, y, jnp.exp(jnp.minimum(y, 0.0)) - 1.0)

  wspec = pl.BlockSpec((F, F), lambda i: (0, 0))
  return pl.pallas_call(
      body,
      out_shape=jax.ShapeDtypeStruct((N_NODES, F), jnp.float32),
      grid=(N_NODES // rb,),
      in_specs=[
          pl.BlockSpec((rb, F), lambda i: (i, 0)),
          pl.BlockSpec((rb, F), lambda i: (i, 0)),
          pl.BlockSpec((NC, rb, F), lambda i: (0, i, 0)),
          wspec, wspec, wspec,
          pl.BlockSpec((1, F), lambda i: (0, 0)),
      ],
      out_specs=pl.BlockSpec((rb, F), lambda i: (i, 0)),
  )(x2d, t1, qparts, wa, wb, wc, bias)


def kernel(x, edge_index, L_vals, W, b):
  x2d = x.reshape(N_NODES, F)
  e = edge_index.shape[1]
  per_tile = -(-e // NW)
  nch = -(-per_tile // K)
  epad = NW * nch * K
  pad = epad - e
  rows = jnp.concatenate([edge_index[0], jnp.zeros((pad,), jnp.int32)])
  cols = jnp.concatenate([edge_index[1], jnp.zeros((pad,), jnp.int32)])
  lv = jnp.concatenate([L_vals, jnp.zeros((pad,), jnp.float32)])
  rows3 = rows.reshape(NW, nch, K)
  cols3 = cols.reshape(NW, nch, K)
  lv3 = jnp.broadcast_to(lv.reshape(NW, nch, K)[..., None],
                         (NW, nch, K, LANES)).reshape(NW, nch, K * LANES)
  zeros2d = jnp.zeros((NPAD, F), jnp.float32)
  xp = jnp.concatenate([x2d, jnp.zeros((NPAD - N_NODES, F), jnp.float32)])

  p1 = _spmm_partials(xp, rows3, cols3, lv3, zeros2d)
  t1 = _merge(p1)
  q = _spmm_partials(t1, rows3, cols3, lv3, zeros2d)

  wp = W.reshape(F, 3, F)
  out = _finish(x2d, t1[:N_NODES], q[:, :N_NODES], wp[:, 0, :], wp[:, 1, :],
                wp[:, 2, :], b.reshape(1, F))
  return out.reshape(1, N_NODES, F)
